# Initial kernel scaffold; baseline (speedup 1.0000x reference)
import jax
import jax.numpy as jnp
from jax.experimental import pallas as pl


def kernel(points, features, mask, _, W1_0, W1_1, W1_2, SC1, W2_0, W2_1, W2_2, SC2, WF, WFC1, BFC1, WFC2, BFC2):
    raise NotImplementedError("write your pallas kernel here")


# speedup vs baseline: 8.2272x; 8.2272x over previous
import functools

import jax
import jax.numpy as jnp
from jax import lax
from jax.experimental import pallas as pl

_EPS = 1e-5
_KNB = 7
_NPT = 128
_TB = 8
_NEG_INF = float('-inf')

_DN_MM = (((1,), (0,)), ((), ()))
_DN_C00 = (((0,), (0,)), ((), ()))
_DN_C11 = (((1,), (1,)), ((), ()))
_DN_C01 = (((0,), (1,)), ((), ()))


def _mm(a, b, dn=_DN_MM):
    return lax.dot_general(a, b, dn, precision=lax.Precision.HIGHEST,
                           preferred_element_type=jnp.float32)


def _first_argmax(m):
    mx = jnp.max(m, axis=1, keepdims=True)
    it = lax.broadcasted_iota(jnp.int32, m.shape, 1)
    return jnp.min(jnp.where(m == mx, it, _NPT), axis=1, keepdims=True)


def _topk_idx(score):
    it = lax.broadcasted_iota(jnp.int32, score.shape, 1)
    ams = []
    m = score
    for r in range(_KNB + 1):
        am = _first_argmax(m)
        ams.append(am)
        if r < _KNB:
            m = jnp.where(it == am, _NEG_INF, m)
    return ams


def _knn_scores(x):
    inner = _mm(x, x, _DN_C00)
    sq = jnp.sum(x * x, axis=0, keepdims=True)
    return 2.0 * inner - sq


def _gather_onehot(idx_col):
    it = lax.broadcasted_iota(jnp.int32, (_KNB * _NPT, _NPT), 1)
    return (idx_col == it).astype(jnp.float32)


def _mean_k(x):
    acc = x[:, 0:_NPT]
    for k in range(1, _KNB):
        acc = acc + x[:, k * _NPT:(k + 1) * _NPT]
    return acc * (1.0 / _KNB)


def _init_zero(refs):
    @pl.when(pl.program_id(0) == 0)
    def _():
        for r in refs:
            r[...] = jnp.zeros_like(r)


def _k0_body(f_ref, s_ref, q_ref):
    acc_s = jnp.zeros((16, 1), jnp.float32)
    acc_q = jnp.zeros((16, 1), jnp.float32)
    for b in range(_TB):
        f = f_ref[b]
        acc_s = acc_s + jnp.sum(f, axis=1, keepdims=True)
        acc_q = acc_q + jnp.sum(f * f, axis=1, keepdims=True)
    _init_zero([s_ref, q_ref])
    s_ref[...] += acc_s
    q_ref[...] += acc_q


def _k1_body(p_ref, f_ref, pw_ref, qw_ref, c_ref, asc_ref, bsc_ref,
             y_ref, s_ref, q_ref, ss_ref, qs_ref):
    scores = []
    for b in range(_TB):
        scores.append(_knn_scores(p_ref[b]))
    ams = _topk_idx(jnp.concatenate(scores, axis=0))
    acc = [jnp.zeros((32, 1), jnp.float32) for _ in range(4)]
    for b in range(_TB):
        idx_col = jnp.concatenate(
            [ams[r][b * _NPT:(b + 1) * _NPT] for r in range(1, _KNB + 1)],
            axis=0)
        onehot = _gather_onehot(idx_col)
        f = f_ref[b]
        h = _mm(pw_ref[...], f)
        g = _mm(qw_ref[...], f)
        gath = _mm(g, onehot, _DN_C11)
        y = gath + jnp.concatenate([h] * _KNB, axis=1) + c_ref[...]
        y_ref[b] = y
        acc[0] = acc[0] + jnp.sum(y, axis=1, keepdims=True)
        acc[1] = acc[1] + jnp.sum(y * y, axis=1, keepdims=True)
        ysc = _mm(asc_ref[...], f) + bsc_ref[...]
        acc[2] = acc[2] + jnp.sum(ysc, axis=1, keepdims=True)
        acc[3] = acc[3] + jnp.sum(ysc * ysc, axis=1, keepdims=True)
    _init_zero([s_ref, q_ref, ss_ref, qs_ref])
    s_ref[...] += acc[0]
    q_ref[...] += acc[1]
    ss_ref[...] += acc[2]
    qs_ref[...] += acc[3]


def _conv_body(c, y_in_ref, sc_ref, bi_ref, w_ref, y_out_ref, s_ref, q_ref):
    acc_s = jnp.zeros((c, 1), jnp.float32)
    acc_q = jnp.zeros((c, 1), jnp.float32)
    for b in range(_TB):
        x = jnp.maximum(sc_ref[...] * y_in_ref[b] + bi_ref[...], 0.0)
        y = _mm(w_ref[...], x)
        y_out_ref[b] = y
        acc_s = acc_s + jnp.sum(y, axis=1, keepdims=True)
        acc_q = acc_q + jnp.sum(y * y, axis=1, keepdims=True)
    _init_zero([s_ref, q_ref])
    s_ref[...] += acc_s
    q_ref[...] += acc_q


def _k4_body(y_ref, f_ref, sc_ref, bi_ref, asc_ref, bsc_ref, s1_ref, b1_ref,
             p2_ref, q2_ref, sc2w_ref,
             o1_ref, y2_ref, s_ref, q_ref, ss_ref, qs_ref):
    o1s = []
    scores = []
    for b in range(_TB):
        x3 = jnp.maximum(sc_ref[...] * y_ref[b] + bi_ref[...], 0.0)
        fts = _mean_k(x3)
        ysc = _mm(asc_ref[...], f_ref[b]) + bsc_ref[...]
        shortcut = s1_ref[...] * ysc + b1_ref[...]
        o1 = jnp.maximum(shortcut + fts, 0.0)
        o1s.append(o1)
        scores.append(_knn_scores(o1))
    ams = _topk_idx(jnp.concatenate(scores, axis=0))
    acc = [jnp.zeros((64, 1), jnp.float32) for _ in range(4)]
    for b in range(_TB):
        o1 = o1s[b]
        o1_ref[b] = o1
        idx_col = jnp.concatenate(
            [ams[r][b * _NPT:(b + 1) * _NPT] for r in range(1, _KNB + 1)],
            axis=0)
        onehot = _gather_onehot(idx_col)
        h2 = _mm(p2_ref[...], o1)
        g2 = _mm(q2_ref[...], o1)
        gath = _mm(g2, onehot, _DN_C11)
        y2 = gath + jnp.concatenate([h2] * _KNB, axis=1)
        y2_ref[b] = y2
        acc[0] = acc[0] + jnp.sum(y2, axis=1, keepdims=True)
        acc[1] = acc[1] + jnp.sum(y2 * y2, axis=1, keepdims=True)
        ysc2 = _mm(sc2w_ref[...], o1)
        acc[2] = acc[2] + jnp.sum(ysc2, axis=1, keepdims=True)
        acc[3] = acc[3] + jnp.sum(ysc2 * ysc2, axis=1, keepdims=True)
    _init_zero([s_ref, q_ref, ss_ref, qs_ref])
    s_ref[...] += acc[0]
    q_ref[...] += acc[1]
    ss_ref[...] += acc[2]
    qs_ref[...] += acc[3]


def _k7_body(y_ref, o1_ref, sc_ref, bi_ref, sc2w_ref, s2_ref, b2_ref,
             wf_ref, yf_ref, s_ref, q_ref):
    acc_s = jnp.zeros((128, 1), jnp.float32)
    acc_q = jnp.zeros((128, 1), jnp.float32)
    for b in range(_TB):
        x = jnp.maximum(sc_ref[...] * y_ref[b] + bi_ref[...], 0.0)
        fts2 = _mean_k(x)
        o1 = o1_ref[b]
        ysc2 = _mm(sc2w_ref[...], o1)
        shortcut = s2_ref[...] * ysc2 + b2_ref[...]
        o2 = jnp.maximum(shortcut + fts2, 0.0)
        z = jnp.concatenate([o1, o2], axis=0)
        yf = _mm(wf_ref[...], z)
        yf_ref[b] = yf
        acc_s = acc_s + jnp.sum(yf, axis=1, keepdims=True)
        acc_q = acc_q + jnp.sum(yf * yf, axis=1, keepdims=True)
    _init_zero([s_ref, q_ref])
    s_ref[...] += acc_s
    q_ref[...] += acc_q


def _k8_body(yf_ref, sc_ref, bi_ref, w1_ref, b1_ref, w2_ref, b2_ref,
             out_ref):
    cols = []
    for b in range(_TB):
        x = jnp.maximum(sc_ref[...] * yf_ref[b] + bi_ref[...], 0.0)
        cols.append(jnp.sum(x, axis=1, keepdims=True) * (1.0 / _NPT))
    v = jnp.concatenate(cols, axis=1)
    h = jnp.maximum(_mm(w1_ref[...], v) + b1_ref[...], 0.0)
    out = _mm(h, w2_ref[...], _DN_C01) + b2_ref[...]
    out_ref[...] = out


def _bspec(shape):
    nd = len(shape)
    return pl.BlockSpec((_TB,) + shape[1:],
                        lambda i, _nd=nd: (i,) + (0,) * (_nd - 1))


def _fspec(shape):
    nd = len(shape)
    return pl.BlockSpec(shape, lambda i, _nd=nd: (0,) * _nd)


def _sds(shape):
    return jax.ShapeDtypeStruct(shape, jnp.float32)


def _scale_bias(s, q, m):
    mean = s / m
    var = q / m - mean * mean
    scale = 1.0 / jnp.sqrt(var + _EPS)
    return scale, -mean * scale


def kernel(points, features, mask, _, W1_0, W1_1, W1_2, SC1, W2_0, W2_1,
           W2_2, SC2, WF, WFC1, BFC1, WFC2, BFC2):
    B = points.shape[0]
    grid = (B // _TB,)

    def call(body, ins, in_specs, outs, out_specs):
        return pl.pallas_call(
            body, grid=grid,
            in_specs=in_specs, out_specs=out_specs,
            out_shape=outs)(*ins)

    s0, q0 = call(_k0_body, [features], [_bspec(features.shape)],
                  [_sds((16, 1)), _sds((16, 1))],
                  [_fspec((16, 1)), _fspec((16, 1))])
    sc0, bi0 = _scale_bias(s0, q0, B * _NPT)
    s0f, b0f = sc0[:, 0], bi0[:, 0]

    wl, wr = W1_0[:, :16], W1_0[:, 16:]
    p1 = (wl - wr) * s0f[None, :]
    q1 = wr * s0f[None, :]
    c1 = (wl @ b0f).reshape(32, 1)
    asc = SC1 * s0f[None, :]
    bsc = (SC1 @ b0f).reshape(32, 1)

    y10, s1, q1s, ssc1, qsc1 = call(
        _k1_body,
        [points, features, p1, q1, c1, asc, bsc],
        [_bspec(points.shape), _bspec(features.shape), _fspec((32, 16)),
         _fspec((32, 16)), _fspec((32, 1)), _fspec((32, 16)),
         _fspec((32, 1))],
        [_sds((B, 32, _KNB * _NPT)), _sds((32, 1)), _sds((32, 1)),
         _sds((32, 1)), _sds((32, 1))],
        [_bspec((B, 32, _KNB * _NPT)), _fspec((32, 1)), _fspec((32, 1)),
         _fspec((32, 1)), _fspec((32, 1))])
    m_e1 = B * _KNB * _NPT

    def conv_stage(y_in, s, q, w, c):
        scl, bi = _scale_bias(s, q, m_e1 if c == 32 else m_e2)
        return call(
            functools.partial(_conv_body, c),
            [y_in, scl, bi, w],
            [_bspec(y_in.shape), _fspec((c, 1)), _fspec((c, 1)),
             _fspec((c, c))],
            [_sds((B, c, _KNB * _NPT)), _sds((c, 1)), _sds((c, 1))],
            [_bspec((B, c, _KNB * _NPT)), _fspec((c, 1)), _fspec((c, 1))])

    m_e2 = B * _KNB * _NPT
    y11, s2, q2s = conv_stage(y10, s1, q1s, W1_1, 32)
    y12, s3, q3s = conv_stage(y11, s2, q2s, W1_2, 32)

    sc12, bi12 = _scale_bias(s3, q3s, m_e1)
    scs1, bis1 = _scale_bias(ssc1, qsc1, B * _NPT)
    p2 = W2_0[:, :32] - W2_0[:, 32:]
    q2w = W2_0[:, 32:]
    o1, y20, s4, q4s, ssc2, qsc2 = call(
        _k4_body,
        [y12, features, sc12, bi12, asc, bsc, scs1, bis1, p2, q2w, SC2],
        [_bspec(y12.shape), _bspec(features.shape), _fspec((32, 1)),
         _fspec((32, 1)), _fspec((32, 16)), _fspec((32, 1)),
         _fspec((32, 1)), _fspec((32, 1)), _fspec((64, 32)),
         _fspec((64, 32)), _fspec((64, 32))],
        [_sds((B, 32, _NPT)), _sds((B, 64, _KNB * _NPT)), _sds((64, 1)),
         _sds((64, 1)), _sds((64, 1)), _sds((64, 1))],
        [_bspec((B, 32, _NPT)), _bspec((B, 64, _KNB * _NPT)),
         _fspec((64, 1)), _fspec((64, 1)), _fspec((64, 1)),
         _fspec((64, 1))])

    y21, s5, q5s = conv_stage(y20, s4, q4s, W2_1, 64)
    y22, s6, q6s = conv_stage(y21, s5, q5s, W2_2, 64)

    sc22, bi22 = _scale_bias(s6, q6s, m_e2)
    scs2, bis2 = _scale_bias(ssc2, qsc2, B * _NPT)
    yf, sf, qf = call(
        _k7_body,
        [y22, o1, sc22, bi22, SC2, scs2, bis2, WF],
        [_bspec(y22.shape), _bspec(o1.shape), _fspec((64, 1)),
         _fspec((64, 1)), _fspec((64, 32)), _fspec((64, 1)),
         _fspec((64, 1)), _fspec((128, 96))],
        [_sds((B, 128, _NPT)), _sds((128, 1)), _sds((128, 1))],
        [_bspec((B, 128, _NPT)), _fspec((128, 1)), _fspec((128, 1))])

    scf, bif = _scale_bias(sf, qf, B * _NPT)
    out = call(
        _k8_body,
        [yf, scf, bif, WFC1, BFC1.reshape(128, 1), WFC2,
         BFC2.reshape(1, 2)],
        [_bspec(yf.shape), _fspec((128, 1)), _fspec((128, 1)),
         _fspec((128, 128)), _fspec((128, 1)), _fspec((2, 128)),
         _fspec((1, 2))],
        _sds((B, 2)),
        _bspec((B, 2)))
    return out
